# Initial kernel scaffold; baseline (speedup 1.0000x reference)
import functools

import jax
import jax.numpy as jnp
from jax import lax
from jax.experimental import pallas as pl
from jax.experimental.pallas import tpu as pltpu
from jax.experimental.pallas import tpu_sc as plsc

N = 10000
D = 128
E = 320000
P = 65536

NC = 2
NS = 16
NW = NC * NS

E_PER_W = E // NW
CHUNK = 80
NCHUNK = E_PER_W // CHUNK

ROWS_PER_TILE = N // NS
ZROWS = 25

CW = 16

PAIRS_PER_W = 2 * P // NW
PCHUNK = 128
NPCHUNK = PAIRS_PER_W // PCHUNK

_MESH = plsc.VectorSubcoreMesh(core_axis_name="c", subcore_axis_name="s")


def _seg_sum_body(with_counts, x_hbm, src_hbm, dst_hbm, *refs):
  if with_counts:
    (acc_out, cnt_out, src_idx, dst_idx, rows, zbuf, acc_sh, sem,
     ones_b, zbuf16, cnt_sh) = refs
  else:
    (acc_out, src_idx, dst_idx, rows, zbuf, acc_sh, sem) = refs
  c = lax.axis_index("c")
  s = lax.axis_index("s")
  wid = s * NC + c

  zero16 = jnp.zeros((16,), jnp.float32)
  for i in range(ZROWS):
    for j in range(D // 16):
      zbuf[i, pl.ds(j * 16, 16)] = zero16
  if with_counts:
    one16 = jnp.ones((16,), jnp.float32)
    for i in range(ZROWS):
      zbuf16[i, :] = zero16
    for i in range(CHUNK):
      ones_b[i, :] = one16

  row0 = s * ROWS_PER_TILE

  def zloop(g, carry):
    pltpu.sync_copy(zbuf, acc_sh.at[pl.ds(row0 + g * ZROWS, ZROWS)])
    if with_counts:
      pltpu.sync_copy(zbuf16, cnt_sh.at[pl.ds(row0 + g * ZROWS, ZROWS)])
    return carry

  lax.fori_loop(0, ROWS_PER_TILE // ZROWS, zloop, 0)
  plsc.subcore_barrier()

  base = wid * E_PER_W

  def eloop(g, carry):
    off = base + g * CHUNK
    pltpu.sync_copy(src_hbm.at[pl.ds(off, CHUNK)], src_idx)
    pltpu.sync_copy(dst_hbm.at[pl.ds(off, CHUNK)], dst_idx)
    pltpu.async_copy(x_hbm.at[src_idx], rows, sem).wait()
    pltpu.sync_copy(rows, acc_sh.at[dst_idx], add=True)
    if with_counts:
      pltpu.sync_copy(ones_b, cnt_sh.at[dst_idx], add=True)
    return carry

  lax.fori_loop(0, NCHUNK, eloop, 0)
  plsc.subcore_barrier()

  pltpu.sync_copy(acc_sh.at[pl.ds(row0, ROWS_PER_TILE)],
                  acc_out.at[c, pl.ds(row0, ROWS_PER_TILE)])
  if with_counts:
    pltpu.sync_copy(cnt_sh.at[pl.ds(row0, ROWS_PER_TILE)],
                    cnt_out.at[c, pl.ds(row0, ROWS_PER_TILE)])


def _make_seg_sum(with_counts):
  out_type = [jax.ShapeDtypeStruct((NC, N, D), jnp.float32)]
  scratch = [
      pltpu.VMEM((CHUNK,), jnp.int32),
      pltpu.VMEM((CHUNK,), jnp.int32),
      pltpu.VMEM((CHUNK, D), jnp.float32),
      pltpu.VMEM((ZROWS, D), jnp.float32),
      pltpu.VMEM_SHARED((N, D), jnp.float32),
      pltpu.SemaphoreType.DMA,
  ]
  if with_counts:
    out_type.append(jax.ShapeDtypeStruct((NC, N, CW), jnp.float32))
    scratch += [
        pltpu.VMEM((CHUNK, CW), jnp.float32),
        pltpu.VMEM((ZROWS, CW), jnp.float32),
        pltpu.VMEM_SHARED((N, CW), jnp.float32),
    ]
  return pl.kernel(
      functools.partial(_seg_sum_body, with_counts),
      out_type=out_type,
      mesh=_MESH,
      scratch_types=scratch,
  )


_seg_sum_counts = _make_seg_sum(True)
_seg_sum = _make_seg_sum(False)


def _pair_gather_body(h_hbm, pidx_hbm, out_hbm, idx, rows, sem):
  c = lax.axis_index("c")
  s = lax.axis_index("s")
  wid = s * NC + c
  base = wid * PAIRS_PER_W

  def gloop(g, carry):
    off = base + g * PCHUNK
    pltpu.sync_copy(pidx_hbm.at[pl.ds(off, PCHUNK)], idx)
    pltpu.async_copy(h_hbm.at[idx], rows, sem).wait()
    pltpu.sync_copy(rows, out_hbm.at[pl.ds(off, PCHUNK)])
    return carry

  lax.fori_loop(0, NPCHUNK, gloop, 0)


_pair_gather = pl.kernel(
    _pair_gather_body,
    out_type=jax.ShapeDtypeStruct((2 * P, D), jnp.float32),
    mesh=_MESH,
    scratch_types=[
        pltpu.VMEM((PCHUNK,), jnp.int32),
        pltpu.VMEM((PCHUNK, D), jnp.float32),
        pltpu.SemaphoreType.DMA,
    ],
)


BN = 1000


def _tc_layer_kernel(relu, acc_ref, cnt_ref, x_ref, wl_ref, wr_ref, b_ref,
                     o_ref):
  cnt = cnt_ref[0] + cnt_ref[1]
  recip = 1.0 / jnp.maximum(cnt[:, 0:1], 1.0)
  agg = (acc_ref[0] + acc_ref[1]) * recip
  h = (jnp.dot(agg, wl_ref[...], preferred_element_type=jnp.float32)
       + jnp.dot(x_ref[...], wr_ref[...], preferred_element_type=jnp.float32)
       + b_ref[...])
  o_ref[...] = jnp.maximum(h, 0.0) if relu else h


def _tc_layer(acc, cnt, x, wl, wr, b, relu):
  return pl.pallas_call(
      functools.partial(_tc_layer_kernel, relu),
      grid=(N // BN,),
      in_specs=[
          pl.BlockSpec((NC, BN, D), lambda i: (0, i, 0)),
          pl.BlockSpec((NC, BN, CW), lambda i: (0, i, 0)),
          pl.BlockSpec((BN, D), lambda i: (i, 0)),
          pl.BlockSpec((D, D), lambda i: (0, 0)),
          pl.BlockSpec((D, D), lambda i: (0, 0)),
          pl.BlockSpec((1, D), lambda i: (0, 0)),
      ],
      out_specs=pl.BlockSpec((BN, D), lambda i: (i, 0)),
      out_shape=jax.ShapeDtypeStruct((N, D), jnp.float32),
  )(acc, cnt, x, wl, wr, b)


BP = 2048


def _tc_mlp_kernel(hs_ref, hd_ref, wt_ref, wb_ref, bp1_ref, wp2_ref, bp2_ref,
                   o_ref):
  z = (jnp.dot(hs_ref[...], wt_ref[...], preferred_element_type=jnp.float32)
       + jnp.dot(hd_ref[...], wb_ref[...], preferred_element_type=jnp.float32)
       + bp1_ref[...])
  z = jnp.maximum(z, 0.0)
  t = jnp.sum(z * wp2_ref[...], axis=1, keepdims=True) + bp2_ref[...]
  o_ref[...] = 1.0 / (1.0 + jnp.exp(-t))


def _tc_mlp(hcat, wt, wb, bp1, wp2_row, bp2):
  return pl.pallas_call(
      _tc_mlp_kernel,
      grid=(P // BP,),
      in_specs=[
          pl.BlockSpec((BP, D), lambda i: (i, 0)),
          pl.BlockSpec((BP, D), lambda i: (P // BP + i, 0)),
          pl.BlockSpec((D, D), lambda i: (0, 0)),
          pl.BlockSpec((D, D), lambda i: (0, 0)),
          pl.BlockSpec((1, D), lambda i: (0, 0)),
          pl.BlockSpec((1, D), lambda i: (0, 0)),
          pl.BlockSpec((1, 1), lambda i: (0, 0)),
      ],
      out_specs=pl.BlockSpec((BP, 1), lambda i: (i, 0)),
      out_shape=jax.ShapeDtypeStruct((P, 1), jnp.float32),
  )(hcat, hcat, wt, wb, bp1, wp2_row, bp2)


def kernel(x, edge_index, edge_pairs, W1l, W1r, b1, W2l, W2r, b2, Wp1, bp1,
           Wp2, bp2):
  src = edge_index[0]
  dst = edge_index[1]
  pidx = edge_pairs.reshape(2 * P)

  acc1, cnt = _seg_sum_counts(x, src, dst)
  h1 = _tc_layer(acc1, cnt, x, W1l, W1r, b1.reshape(1, D), relu=True)
  acc2 = _seg_sum(h1, src, dst)
  h2 = _tc_layer(acc2, cnt, h1, W2l, W2r, b2.reshape(1, D), relu=False)
  hcat = _pair_gather(h2, pidx)
  out = _tc_mlp(hcat, Wp1[:D], Wp1[D:], bp1.reshape(1, D),
                Wp2.reshape(1, D), bp2.reshape(1, 1))
  return out[:, 0]


# speedup vs baseline: 4.3751x; 4.3751x over previous
import functools

import jax
import jax.numpy as jnp
from jax import lax
from jax.experimental import pallas as pl
from jax.experimental.pallas import tpu as pltpu
from jax.experimental.pallas import tpu_sc as plsc

N = 10000
D = 128
E = 320000
P = 65536

NC = 2
NS = 16
NW = NC * NS

E_PER_W = E // NW
CHUNK = 80
NCHUNK = E_PER_W // CHUNK

ROWS_T = 624
ZROWS = 16

PAIRS_PER_W = 2 * P // NW
PCHUNK = 128
NPCHUNK = PAIRS_PER_W // PCHUNK

_MESH = plsc.VectorSubcoreMesh(core_axis_name="c", subcore_axis_name="s")


def _seg_sum_body(with_counts, x_hbm, src_hbm, dst_hbm, z_hbm, *refs):
  if with_counts:
    (acc_out, cnt_out, src_idx, dst_idx, rows, zbuf, acc_sh, sem, hist) = refs
  else:
    (acc_out, src_idx, dst_idx, rows, zbuf, acc_sh, sem) = refs
  c = lax.axis_index("c")
  s = lax.axis_index("s")
  wid = s * NC + c

  zero16 = jnp.zeros((16,), jnp.float32)
  one16 = jnp.ones((16,), jnp.float32)
  pltpu.sync_copy(z_hbm, zbuf)

  if with_counts:
    def hloop(i, carry):
      hist[pl.ds(i * 16, 16)] = zero16
      return carry

    lax.fori_loop(0, N // 16, hloop, 0)

  row0 = s * ROWS_T

  def zloop(g, carry):
    pltpu.sync_copy(zbuf, acc_sh.at[pl.ds(row0 + g * ZROWS, ZROWS)])
    return carry

  lax.fori_loop(0, ROWS_T // ZROWS, zloop, 0)

  @pl.when(s == NS - 1)
  def _zero_tail():
    pltpu.sync_copy(zbuf, acc_sh.at[pl.ds(N - ZROWS, ZROWS)])

  plsc.subcore_barrier()

  base = wid * E_PER_W

  def eloop(g, carry):
    off = base + g * CHUNK
    pltpu.sync_copy(src_hbm.at[pl.ds(off, CHUNK)], src_idx)
    pltpu.sync_copy(dst_hbm.at[pl.ds(off, CHUNK)], dst_idx)
    pltpu.async_copy(x_hbm.at[src_idx], rows, sem).wait()
    pltpu.sync_copy(rows, acc_sh.at[dst_idx], add=True)
    if with_counts:
      for j in range(CHUNK // 16):
        dv = dst_idx[pl.ds(j * 16, 16)]
        plsc.addupdate_scatter(hist, [dv], one16)
    return carry

  lax.fori_loop(0, NCHUNK, eloop, 0)
  plsc.subcore_barrier()

  pltpu.sync_copy(acc_sh.at[pl.ds(row0, ROWS_T)],
                  acc_out.at[c, pl.ds(row0, ROWS_T)])
  if with_counts:
    pltpu.sync_copy(hist, cnt_out.at[wid])

  @pl.when(s == NS - 1)
  def _write_tail():
    pltpu.sync_copy(acc_sh.at[pl.ds(N - ZROWS, ZROWS)],
                    acc_out.at[c, pl.ds(N - ZROWS, ZROWS)])


def _make_seg_sum(with_counts):
  acc_type = jax.ShapeDtypeStruct((NC, N, D), jnp.float32)
  out_type = [acc_type] if with_counts else acc_type
  scratch = [
      pltpu.VMEM((CHUNK,), jnp.int32),
      pltpu.VMEM((CHUNK,), jnp.int32),
      pltpu.VMEM((CHUNK, D), jnp.float32),
      pltpu.VMEM((ZROWS, D), jnp.float32),
      pltpu.VMEM_SHARED((N, D), jnp.float32),
      pltpu.SemaphoreType.DMA,
  ]
  if with_counts:
    out_type.append(jax.ShapeDtypeStruct((NW, N), jnp.float32))
    scratch.append(pltpu.VMEM((N,), jnp.float32))
  return pl.kernel(
      functools.partial(_seg_sum_body, with_counts),
      out_type=out_type,
      mesh=_MESH,
      compiler_params=pltpu.CompilerParams(needs_layout_passes=False)
      if with_counts else None,
      scratch_types=scratch,
  )


_seg_sum_counts = _make_seg_sum(True)
_seg_sum = _make_seg_sum(False)


def _pair_gather_body(h_hbm, pidx_hbm, out_hbm, idx, rows, sem):
  c = lax.axis_index("c")
  s = lax.axis_index("s")
  wid = s * NC + c
  base = wid * PAIRS_PER_W

  def gloop(g, carry):
    off = base + g * PCHUNK
    pltpu.sync_copy(pidx_hbm.at[pl.ds(off, PCHUNK)], idx)
    pltpu.async_copy(h_hbm.at[idx], rows, sem).wait()
    pltpu.sync_copy(rows, out_hbm.at[pl.ds(off, PCHUNK)])
    return carry

  lax.fori_loop(0, NPCHUNK, gloop, 0)


_pair_gather = pl.kernel(
    _pair_gather_body,
    out_type=jax.ShapeDtypeStruct((2 * P, D), jnp.float32),
    mesh=_MESH,
    scratch_types=[
        pltpu.VMEM((PCHUNK,), jnp.int32),
        pltpu.VMEM((PCHUNK, D), jnp.float32),
        pltpu.SemaphoreType.DMA,
    ],
)


BN = 1000


def _tc_layer_kernel(relu, acc_ref, cnt_ref, x_ref, wl_ref, wr_ref, b_ref,
                     o_ref):
  cnt = cnt_ref[0]
  for w in range(1, NW):
    cnt = cnt + cnt_ref[w]
  recip = 1.0 / jnp.maximum(cnt, 1.0)
  agg = (acc_ref[0] + acc_ref[1]) * recip
  h = (jnp.dot(agg, wl_ref[...], preferred_element_type=jnp.float32)
       + jnp.dot(x_ref[...], wr_ref[...], preferred_element_type=jnp.float32)
       + b_ref[...])
  o_ref[...] = jnp.maximum(h, 0.0) if relu else h


def _tc_layer(acc, cnt, x, wl, wr, b, relu):
  return pl.pallas_call(
      functools.partial(_tc_layer_kernel, relu),
      grid=(N // BN,),
      in_specs=[
          pl.BlockSpec((NC, BN, D), lambda i: (0, i, 0)),
          pl.BlockSpec((NW, BN, 1), lambda i: (0, i, 0)),
          pl.BlockSpec((BN, D), lambda i: (i, 0)),
          pl.BlockSpec((D, D), lambda i: (0, 0)),
          pl.BlockSpec((D, D), lambda i: (0, 0)),
          pl.BlockSpec((1, D), lambda i: (0, 0)),
      ],
      out_specs=pl.BlockSpec((BN, D), lambda i: (i, 0)),
      out_shape=jax.ShapeDtypeStruct((N, D), jnp.float32),
  )(acc, cnt, x, wl, wr, b)


BP = 2048


def _tc_mlp_kernel(hs_ref, hd_ref, wt_ref, wb_ref, bp1_ref, wp2_ref, bp2_ref,
                   o_ref):
  z = (jnp.dot(hs_ref[...], wt_ref[...], preferred_element_type=jnp.float32)
       + jnp.dot(hd_ref[...], wb_ref[...], preferred_element_type=jnp.float32)
       + bp1_ref[...])
  z = jnp.maximum(z, 0.0)
  t = jnp.sum(z * wp2_ref[...], axis=1, keepdims=True) + bp2_ref[...]
  o_ref[...] = 1.0 / (1.0 + jnp.exp(-t))


def _tc_mlp(hcat, wt, wb, bp1, wp2_row, bp2):
  return pl.pallas_call(
      _tc_mlp_kernel,
      grid=(P // BP,),
      in_specs=[
          pl.BlockSpec((BP, D), lambda i: (i, 0)),
          pl.BlockSpec((BP, D), lambda i: (P // BP + i, 0)),
          pl.BlockSpec((D, D), lambda i: (0, 0)),
          pl.BlockSpec((D, D), lambda i: (0, 0)),
          pl.BlockSpec((1, D), lambda i: (0, 0)),
          pl.BlockSpec((1, D), lambda i: (0, 0)),
          pl.BlockSpec((1, 1), lambda i: (0, 0)),
      ],
      out_specs=pl.BlockSpec((BP, 1), lambda i: (i, 0)),
      out_shape=jax.ShapeDtypeStruct((P, 1), jnp.float32),
  )(hcat, hcat, wt, wb, bp1, wp2_row, bp2)


def kernel(x, edge_index, edge_pairs, W1l, W1r, b1, W2l, W2r, b2, Wp1, bp1,
           Wp2, bp2):
  src = edge_index[0]
  dst = edge_index[1]
  pidx = edge_pairs.reshape(2 * P)

  zeros = jnp.zeros((ZROWS, D), jnp.float32)
  acc1, cnt = _seg_sum_counts(x, src, dst, zeros)
  cnt = cnt.reshape(NW, N, 1)
  h1 = _tc_layer(acc1, cnt, x, W1l, W1r, b1.reshape(1, D), relu=True)
  acc2 = _seg_sum(h1, src, dst, zeros)
  h2 = _tc_layer(acc2, cnt, h1, W2l, W2r, b2.reshape(1, D), relu=False)
  hcat = _pair_gather(h2, pidx)
  out = _tc_mlp(hcat, Wp1[:D], Wp1[D:], bp1.reshape(1, D),
                Wp2.reshape(1, D), bp2.reshape(1, 1))
  return out[:, 0]


# speedup vs baseline: 7.6323x; 1.7445x over previous
import functools

import jax
import jax.numpy as jnp
from jax import lax
from jax.experimental import pallas as pl
from jax.experimental.pallas import tpu as pltpu
from jax.experimental.pallas import tpu_sc as plsc

N = 10000
D = 128
E = 320000
P = 65536

NC = 2
NS = 16
NW = NC * NS

E_PER_W = E // NW
CHUNK = 80
NCHUNK = E_PER_W // CHUNK

ROWS_T = 624
ZROWS = 16

PAIRS_PER_W = 2 * P // NW
PCHUNK = 128
NPCHUNK = PAIRS_PER_W // PCHUNK

_MESH = plsc.VectorSubcoreMesh(core_axis_name="c", subcore_axis_name="s")


def _seg_sum_body(with_counts, x_hbm, edge_hbm, z_hbm, *refs):
  if with_counts:
    (acc_out, cnt_out, src_v, dst_v, r0, r1, acc_sh, s0, s1, hist) = refs
  else:
    (acc_out, src_v, dst_v, r0, r1, acc_sh, s0, s1) = refs
  c = lax.axis_index("c")
  s = lax.axis_index("s")
  wid = s * NC + c
  base = wid * E_PER_W

  pltpu.sync_copy(edge_hbm.at[pl.ds(base, E_PER_W)], src_v)
  pltpu.sync_copy(edge_hbm.at[pl.ds(E + base, E_PER_W)], dst_v)

  zero16 = jnp.zeros((16,), jnp.float32)
  one16 = jnp.ones((16,), jnp.float32)

  if with_counts:
    def hloop(i, carry):
      hist[pl.ds(i * 16, 16)] = zero16
      return carry

    lax.fori_loop(0, N // 16, hloop, 0)

  row0 = s * ROWS_T
  pltpu.sync_copy(z_hbm, acc_sh.at[pl.ds(row0, ROWS_T)])

  @pl.when(s == NS - 1)
  def _zero_tail():
    pltpu.sync_copy(z_hbm.at[pl.ds(0, ZROWS)], acc_sh.at[pl.ds(N - ZROWS, ZROWS)])

  plsc.subcore_barrier()

  def gstart(g, rbuf, sem):
    pltpu.async_copy(x_hbm.at[src_v.at[pl.ds(g * CHUNK, CHUNK)]], rbuf, sem)

  def gwait(g, rbuf, sem):
    pltpu.make_async_copy(x_hbm.at[src_v.at[pl.ds(g * CHUNK, CHUNK)]],
                          rbuf, sem).wait()

  def scat(g, rbuf):
    pltpu.sync_copy(rbuf, acc_sh.at[dst_v.at[pl.ds(g * CHUNK, CHUNK)]],
                    add=True)
    if with_counts:
      for j in range(CHUNK // 16):
        dv = dst_v[pl.ds(g * CHUNK + j * 16, 16)]
        plsc.addupdate_scatter(hist, [dv], one16)

  gstart(0, r0, s0)

  def eloop(i, carry):
    g0 = 2 * i

    @pl.when(g0 + 1 < NCHUNK)
    def _():
      gstart(g0 + 1, r1, s1)

    gwait(g0, r0, s0)
    scat(g0, r0)

    @pl.when(g0 + 2 < NCHUNK)
    def _():
      gstart(g0 + 2, r0, s0)

    @pl.when(g0 + 1 < NCHUNK)
    def _():
      gwait(g0 + 1, r1, s1)
      scat(g0 + 1, r1)

    return carry

  lax.fori_loop(0, (NCHUNK + 1) // 2, eloop, 0)
  plsc.subcore_barrier()

  pltpu.sync_copy(acc_sh.at[pl.ds(row0, ROWS_T)],
                  acc_out.at[c, pl.ds(row0, ROWS_T)])
  if with_counts:
    pltpu.sync_copy(hist, cnt_out.at[wid])

  @pl.when(s == NS - 1)
  def _write_tail():
    pltpu.sync_copy(acc_sh.at[pl.ds(N - ZROWS, ZROWS)],
                    acc_out.at[c, pl.ds(N - ZROWS, ZROWS)])


def _make_seg_sum(with_counts):
  acc_type = jax.ShapeDtypeStruct((NC, N, D), jnp.float32)
  out_type = [acc_type] if with_counts else acc_type
  scratch = [
      pltpu.VMEM((E_PER_W,), jnp.int32),
      pltpu.VMEM((E_PER_W,), jnp.int32),
      pltpu.VMEM((CHUNK, D), jnp.float32),
      pltpu.VMEM((CHUNK, D), jnp.float32),
      pltpu.VMEM_SHARED((N, D), jnp.float32),
      pltpu.SemaphoreType.DMA,
      pltpu.SemaphoreType.DMA,
  ]
  if with_counts:
    out_type.append(jax.ShapeDtypeStruct((NW, N), jnp.float32))
    scratch.append(pltpu.VMEM((N,), jnp.float32))
  return pl.kernel(
      functools.partial(_seg_sum_body, with_counts),
      out_type=out_type,
      mesh=_MESH,
      compiler_params=pltpu.CompilerParams(needs_layout_passes=False)
      if with_counts else None,
      scratch_types=scratch,
  )


_seg_sum_counts = _make_seg_sum(True)
_seg_sum = _make_seg_sum(False)


def _pair_gather_body(h_hbm, pidx_hbm, out_hbm, idx_v, r0, r1, s0, s1):
  c = lax.axis_index("c")
  s = lax.axis_index("s")
  wid = s * NC + c
  base = wid * PAIRS_PER_W
  pltpu.sync_copy(pidx_hbm.at[pl.ds(base, PAIRS_PER_W)], idx_v)

  def gstart(g, rbuf, sem):
    pltpu.async_copy(h_hbm.at[idx_v.at[pl.ds(g * PCHUNK, PCHUNK)]], rbuf, sem)

  def gwait(g, rbuf, sem):
    pltpu.make_async_copy(h_hbm.at[idx_v.at[pl.ds(g * PCHUNK, PCHUNK)]],
                          rbuf, sem).wait()

  def wout(g, rbuf):
    pltpu.sync_copy(rbuf, out_hbm.at[pl.ds(base + g * PCHUNK, PCHUNK)])

  gstart(0, r0, s0)

  def gloop(i, carry):
    g0 = 2 * i

    @pl.when(g0 + 1 < NPCHUNK)
    def _():
      gstart(g0 + 1, r1, s1)

    gwait(g0, r0, s0)
    wout(g0, r0)

    @pl.when(g0 + 2 < NPCHUNK)
    def _():
      gstart(g0 + 2, r0, s0)

    @pl.when(g0 + 1 < NPCHUNK)
    def _():
      gwait(g0 + 1, r1, s1)
      wout(g0 + 1, r1)

    return carry

  lax.fori_loop(0, (NPCHUNK + 1) // 2, gloop, 0)


_pair_gather = pl.kernel(
    _pair_gather_body,
    out_type=jax.ShapeDtypeStruct((2 * P, D), jnp.float32),
    mesh=_MESH,
    scratch_types=[
        pltpu.VMEM((PAIRS_PER_W,), jnp.int32),
        pltpu.VMEM((PCHUNK, D), jnp.float32),
        pltpu.VMEM((PCHUNK, D), jnp.float32),
        pltpu.SemaphoreType.DMA,
        pltpu.SemaphoreType.DMA,
    ],
)


BN = 1000


def _tc_layer_kernel(relu, acc_ref, cnt_ref, x_ref, wl_ref, wr_ref, b_ref,
                     o_ref):
  cnt = cnt_ref[0]
  for w in range(1, NW):
    cnt = cnt + cnt_ref[w]
  recip = 1.0 / jnp.maximum(cnt, 1.0)
  agg = (acc_ref[0] + acc_ref[1]) * recip
  h = (jnp.dot(agg, wl_ref[...], preferred_element_type=jnp.float32)
       + jnp.dot(x_ref[...], wr_ref[...], preferred_element_type=jnp.float32)
       + b_ref[...])
  o_ref[...] = jnp.maximum(h, 0.0) if relu else h


def _tc_layer(acc, cnt, x, wl, wr, b, relu):
  return pl.pallas_call(
      functools.partial(_tc_layer_kernel, relu),
      grid=(N // BN,),
      in_specs=[
          pl.BlockSpec((NC, BN, D), lambda i: (0, i, 0)),
          pl.BlockSpec((NW, BN, 1), lambda i: (0, i, 0)),
          pl.BlockSpec((BN, D), lambda i: (i, 0)),
          pl.BlockSpec((D, D), lambda i: (0, 0)),
          pl.BlockSpec((D, D), lambda i: (0, 0)),
          pl.BlockSpec((1, D), lambda i: (0, 0)),
      ],
      out_specs=pl.BlockSpec((BN, D), lambda i: (i, 0)),
      out_shape=jax.ShapeDtypeStruct((N, D), jnp.float32),
  )(acc, cnt, x, wl, wr, b)


BP = 2048


def _tc_mlp_kernel(hs_ref, hd_ref, wt_ref, wb_ref, bp1_ref, wp2_ref, bp2_ref,
                   o_ref):
  z = (jnp.dot(hs_ref[...], wt_ref[...], preferred_element_type=jnp.float32)
       + jnp.dot(hd_ref[...], wb_ref[...], preferred_element_type=jnp.float32)
       + bp1_ref[...])
  z = jnp.maximum(z, 0.0)
  t = jnp.sum(z * wp2_ref[...], axis=1, keepdims=True) + bp2_ref[...]
  o_ref[...] = 1.0 / (1.0 + jnp.exp(-t))


def _tc_mlp(hcat, wt, wb, bp1, wp2_row, bp2):
  return pl.pallas_call(
      _tc_mlp_kernel,
      grid=(P // BP,),
      in_specs=[
          pl.BlockSpec((BP, D), lambda i: (i, 0)),
          pl.BlockSpec((BP, D), lambda i: (P // BP + i, 0)),
          pl.BlockSpec((D, D), lambda i: (0, 0)),
          pl.BlockSpec((D, D), lambda i: (0, 0)),
          pl.BlockSpec((1, D), lambda i: (0, 0)),
          pl.BlockSpec((1, D), lambda i: (0, 0)),
          pl.BlockSpec((1, 1), lambda i: (0, 0)),
      ],
      out_specs=pl.BlockSpec((BP, 1), lambda i: (i, 0)),
      out_shape=jax.ShapeDtypeStruct((P, 1), jnp.float32),
  )(hcat, hcat, wt, wb, bp1, wp2_row, bp2)


def kernel(x, edge_index, edge_pairs, W1l, W1r, b1, W2l, W2r, b2, Wp1, bp1,
           Wp2, bp2):
  edge_flat = edge_index.reshape(2 * E)
  pidx = edge_pairs.reshape(2 * P)

  zeros = jnp.zeros((ROWS_T, D), jnp.float32)
  acc1, cnt = _seg_sum_counts(x, edge_flat, zeros)
  cnt = cnt.reshape(NW, N, 1)
  h1 = _tc_layer(acc1, cnt, x, W1l, W1r, b1.reshape(1, D), relu=True)
  acc2 = _seg_sum(h1, edge_flat, zeros)
  h2 = _tc_layer(acc2, cnt, h1, W2l, W2r, b2.reshape(1, D), relu=False)
  hcat = _pair_gather(h2, pidx)
  out = _tc_mlp(hcat, Wp1[:D], Wp1[D:], bp1.reshape(1, D),
                Wp2.reshape(1, D), bp2.reshape(1, 1))
  return out[:, 0]
